# baseline (speedup 1.0000x reference)
import functools
import jax
import jax.numpy as jnp
from jax import lax
from jax.experimental import pallas as pl
from jax.experimental.pallas import tpu as pltpu
from jax.experimental.pallas import tpu_sc as plsc

N = 10000
NP = 10240
E = 320000
IN_DIM = 128
H1 = 32
H2 = 64
ACT = 64

NC, NS = 2, 16
NW = NC * NS
CH = 128
NCH_W = 80
NCHT = NW * NCH_W + 2
EPA = NCHT * CH
RPS = NP // NS

_MESH = plsc.VectorSubcoreMesh(core_axis_name="c", subcore_axis_name="s")
_SC_PARAMS = pltpu.CompilerParams(use_tc_tiling_on_sc=False)


@functools.partial(
    pl.kernel,
    out_type=jax.ShapeDtypeStruct((NC, NP), jnp.float32),
    mesh=_MESH,
    compiler_params=_SC_PARAMS,
    scratch_types=[
        pltpu.VMEM((NCH_W, CH), jnp.int32),
        pltpu.VMEM((CH,), jnp.float32),
        pltpu.VMEM_SHARED((NP,), jnp.float32),
    ],
)
def _sc_deg(dst_hbm, zeros_hbm, ones_hbm, out_hbm, dst_all, ones_v, acc_sh):
    c = lax.axis_index("c")
    s = lax.axis_index("s")
    w = s * NC + c

    pltpu.sync_copy(zeros_hbm.at[pl.ds(s * RPS, RPS)],
                    acc_sh.at[pl.ds(s * RPS, RPS)])
    pltpu.sync_copy(dst_hbm.at[pl.ds(w * NCH_W, NCH_W)], dst_all)
    pltpu.sync_copy(ones_hbm, ones_v)
    plsc.subcore_barrier()

    def body(i, carry):
        pltpu.sync_copy(ones_v, acc_sh.at[dst_all.at[i]], add=True)
        return carry

    lax.fori_loop(0, NCH_W, body, 0)
    plsc.subcore_barrier()
    pltpu.sync_copy(acc_sh.at[pl.ds(s * RPS, RPS)],
                    out_hbm.at[c, pl.ds(s * RPS, RPS)])


def _make_sc_agg(F):
    @functools.partial(
        pl.kernel,
        out_type=jax.ShapeDtypeStruct((NC, NP, F), jnp.float32),
        mesh=_MESH,
        compiler_params=_SC_PARAMS,
        scratch_types=[
            pltpu.VMEM((NCH_W + 2, CH), jnp.int32),
            pltpu.VMEM((NCH_W, CH), jnp.int32),
            pltpu.VMEM((CH, F), jnp.float32),
            pltpu.VMEM((CH, F), jnp.float32),
            pltpu.VMEM_SHARED((NP, F), jnp.float32),
            pltpu.SemaphoreType.DMA,
            pltpu.SemaphoreType.DMA,
        ],
    )
    def _sc_agg(g_hbm, src_hbm, dst_hbm, zeros_hbm, out_hbm,
                src_all, dst_all, rows_a, rows_b, acc_sh, sem_a, sem_b):
        c = lax.axis_index("c")
        s = lax.axis_index("s")
        w = s * NC + c

        rr = pl.ds(s * RPS, RPS)

        @pl.when(c == 0)
        def _():
            pltpu.sync_copy(g_hbm.at[rr], acc_sh.at[rr])

        @pl.when(c == 1)
        def _():
            pltpu.sync_copy(zeros_hbm.at[rr], acc_sh.at[rr])

        base = w * NCH_W
        pltpu.sync_copy(src_hbm.at[pl.ds(base, NCH_W + 2)], src_all)
        pltpu.sync_copy(dst_hbm.at[pl.ds(base, NCH_W)], dst_all)
        plsc.subcore_barrier()

        pltpu.async_copy(g_hbm.at[src_all.at[0]], rows_a, sem_a)
        pltpu.async_copy(g_hbm.at[src_all.at[1]], rows_b, sem_b)

        def body(j, carry):
            ci = 2 * j
            pltpu.make_async_copy(g_hbm.at[pl.ds(0, CH)], rows_a, sem_a).wait()
            pltpu.sync_copy(rows_a, acc_sh.at[dst_all.at[ci]], add=True)
            pltpu.async_copy(g_hbm.at[src_all.at[ci + 2]], rows_a, sem_a)
            pltpu.make_async_copy(g_hbm.at[pl.ds(0, CH)], rows_b, sem_b).wait()
            pltpu.sync_copy(rows_b, acc_sh.at[dst_all.at[ci + 1]], add=True)
            pltpu.async_copy(g_hbm.at[src_all.at[ci + 3]], rows_b, sem_b)
            return carry

        lax.fori_loop(0, NCH_W // 2, body, 0)
        pltpu.make_async_copy(g_hbm.at[pl.ds(0, CH)], rows_a, sem_a).wait()
        pltpu.make_async_copy(g_hbm.at[pl.ds(0, CH)], rows_b, sem_b).wait()

        plsc.subcore_barrier()
        pltpu.sync_copy(acc_sh.at[pl.ds(s * RPS, RPS)],
                        out_hbm.at[c, pl.ds(s * RPS, RPS)])

    return _sc_agg


_sc_agg1 = _make_sc_agg(H1)
_sc_agg2 = _make_sc_agg(H2)


def _tc_g1_body(deg_ref, x_ref, w1_ref, g1_ref, dinv_ref):
    deg = deg_ref[0, :] + deg_ref[1, :] + 1.0
    dinv = lax.rsqrt(deg)
    dinv_ref[...] = dinv
    h = jnp.dot(x_ref[...], w1_ref[...], preferred_element_type=jnp.float32)
    g1_ref[:N, :] = h * dinv[:N, None]
    g1_ref[N:, :] = jnp.zeros((NP - N, H1), jnp.float32)


def _tc_g2_body(p0_ref, p1_ref, dinv_ref, b1_ref, w2_ref, g2_ref):
    dinv = dinv_ref[...][:N]
    a = p0_ref[:N, :] + p1_ref[:N, :]
    h = jnp.maximum(a * dinv[:, None] + b1_ref[...], 0.0)
    g2_ref[:N, :] = jnp.dot(h, w2_ref[...],
                            preferred_element_type=jnp.float32) * dinv[:, None]
    g2_ref[N:, :] = jnp.zeros((NP - N, H2), jnp.float32)


def _tc_h2_body(p0_ref, p1_ref, dinv_ref, b2_ref, h2_ref):
    dinv = dinv_ref[...][:N]
    a = p0_ref[:N, :] + p1_ref[:N, :]
    h2_ref[...] = jnp.maximum(a * dinv[:, None] + b2_ref[...], 0.0)


BK = 6400
KSTEPS = (N * H2 // 2) // BK


def _tc_head_body(fe_ref, fo_ref, w_ref, bout_ref, out_ref, acc_ref):
    k = pl.program_id(0)
    f2 = jnp.concatenate([fe_ref[...], fo_ref[...]], axis=0)
    part = jnp.dot(f2, w_ref[...], preferred_element_type=jnp.float32)

    @pl.when(k == 0)
    def _():
        acc_ref[...] = part

    @pl.when(k > 0)
    def _():
        acc_ref[...] += part

    @pl.when(k == KSTEPS - 1)
    def _():
        a = acc_ref[...]
        comb = a[0:1, :] + pltpu.roll(a[1:2, :], 64, 1)
        logits = comb[:, :ACT] + bout_ref[...]
        m = jnp.max(logits, axis=-1, keepdims=True)
        e = jnp.exp(logits - m)
        out_ref[...] = e / jnp.sum(e, axis=-1, keepdims=True)


def kernel(x, ei, W1, b1, W2, b2, Wout, bout):
    src = ei[0].astype(jnp.int32)
    dst = ei[1].astype(jnp.int32)
    npad = EPA - E
    pad_idx = N + (jnp.arange(npad, dtype=jnp.int32) % (NP - N))
    src_p = jnp.concatenate([src, pad_idx]).reshape(NCHT, CH)
    dst_p = jnp.concatenate([dst, pad_idx]).reshape(NCHT, CH)

    zeros_np = jnp.zeros((NP,), jnp.float32)
    ones_ch = jnp.ones((CH,), jnp.float32)

    degp = _sc_deg(dst_p, zeros_np, ones_ch)

    g1, dinv = pl.pallas_call(
        _tc_g1_body,
        out_shape=[jax.ShapeDtypeStruct((NP, H1), jnp.float32),
                   jax.ShapeDtypeStruct((NP,), jnp.float32)],
    )(degp, x, W1)

    zeros_1 = jnp.zeros((NP, H1), jnp.float32)
    acc1 = _sc_agg1(g1, src_p, dst_p, zeros_1)

    g2 = pl.pallas_call(
        _tc_g2_body,
        out_shape=jax.ShapeDtypeStruct((NP, H2), jnp.float32),
    )(acc1[0], acc1[1], dinv, b1.reshape(1, H1), W2)

    zeros_2 = jnp.zeros((NP, H2), jnp.float32)
    acc2 = _sc_agg2(g2, src_p, dst_p, zeros_2)

    h2 = pl.pallas_call(
        _tc_h2_body,
        out_shape=jax.ShapeDtypeStruct((N, H2), jnp.float32),
    )(acc2[0][:N], acc2[1][:N], dinv, b2.reshape(1, H2))

    flat = h2.reshape(N * H2)
    fe = flat[0::2].reshape(1, N * H2 // 2)
    fo = flat[1::2].reshape(1, N * H2 // 2)
    w128 = Wout.reshape(N * H2 // 2, 2 * ACT)
    probs = pl.pallas_call(
        _tc_head_body,
        grid=(KSTEPS,),
        in_specs=[
            pl.BlockSpec((1, BK), lambda k: (0, k)),
            pl.BlockSpec((1, BK), lambda k: (0, k)),
            pl.BlockSpec((BK, 2 * ACT), lambda k: (k, 0)),
            pl.BlockSpec((1, ACT), lambda k: (0, 0)),
        ],
        out_specs=pl.BlockSpec((1, ACT), lambda k: (0, 0)),
        out_shape=jax.ShapeDtypeStruct((1, ACT), jnp.float32),
        scratch_shapes=[pltpu.VMEM((2, 2 * ACT), jnp.float32)],
    )(fe, fo, w128, bout.reshape(1, ACT))

    return probs


# speedup vs baseline: 1.1765x; 1.1765x over previous
import functools
import jax
import jax.numpy as jnp
from jax import lax
from jax.experimental import pallas as pl
from jax.experimental.pallas import tpu as pltpu
from jax.experimental.pallas import tpu_sc as plsc

N = 10000
NP = 10240
E = 320000
IN_DIM = 128
H1 = 32
H2 = 64
ACT = 64

NC, NS = 2, 16
NW = NC * NS
CH = 128
NCH_W = 80
NCHT = NW * NCH_W + 2
EPA = NCHT * CH
RPS = NP // NS

_MESH = plsc.VectorSubcoreMesh(core_axis_name="c", subcore_axis_name="s")
_SC_PARAMS = pltpu.CompilerParams(use_tc_tiling_on_sc=False)


@functools.partial(
    pl.kernel,
    out_type=jax.ShapeDtypeStruct((NC, NP), jnp.float32),
    mesh=_MESH,
    compiler_params=_SC_PARAMS,
    scratch_types=[
        pltpu.VMEM((NCH_W, CH), jnp.int32),
        pltpu.VMEM((CH,), jnp.float32),
        pltpu.VMEM_SHARED((NP,), jnp.float32),
    ],
)
def _sc_deg(dst_hbm, zeros_hbm, ones_hbm, out_hbm, dst_all, ones_v, acc_sh):
    c = lax.axis_index("c")
    s = lax.axis_index("s")
    w = s * NC + c

    pltpu.sync_copy(zeros_hbm.at[pl.ds(s * RPS, RPS)],
                    acc_sh.at[pl.ds(s * RPS, RPS)])
    pltpu.sync_copy(dst_hbm.at[pl.ds(w * NCH_W, NCH_W)], dst_all)
    pltpu.sync_copy(ones_hbm, ones_v)
    plsc.subcore_barrier()

    def body(i, carry):
        pltpu.sync_copy(ones_v, acc_sh.at[dst_all.at[i]], add=True)
        return carry

    lax.fori_loop(0, NCH_W, body, 0)
    plsc.subcore_barrier()
    pltpu.sync_copy(acc_sh.at[pl.ds(s * RPS, RPS)],
                    out_hbm.at[c, pl.ds(s * RPS, RPS)])


def _make_sc_agg(F):
    @functools.partial(
        pl.kernel,
        out_type=jax.ShapeDtypeStruct((NC, NP, F), jnp.float32),
        mesh=_MESH,
        compiler_params=_SC_PARAMS,
        scratch_types=[
            pltpu.VMEM((NCH_W + 2, CH), jnp.int32),
            pltpu.VMEM((NCH_W, CH), jnp.int32),
            pltpu.VMEM((CH, F), jnp.float32),
            pltpu.VMEM((CH, F), jnp.float32),
            pltpu.VMEM_SHARED((NP, F), jnp.float32),
            pltpu.SemaphoreType.DMA,
            pltpu.SemaphoreType.DMA,
        ],
    )
    def _sc_agg(g_hbm, src_hbm, dst_hbm, zeros_hbm, out_hbm,
                src_all, dst_all, rows_a, rows_b, acc_sh, sem_a, sem_b):
        c = lax.axis_index("c")
        s = lax.axis_index("s")
        w = s * NC + c

        rr = pl.ds(s * RPS, RPS)

        @pl.when(c == 0)
        def _():
            pltpu.sync_copy(g_hbm.at[rr], acc_sh.at[rr])

        @pl.when(c == 1)
        def _():
            pltpu.sync_copy(zeros_hbm.at[rr], acc_sh.at[rr])

        base = w * NCH_W
        pltpu.sync_copy(src_hbm.at[pl.ds(base, NCH_W + 2)], src_all)
        pltpu.sync_copy(dst_hbm.at[pl.ds(base, NCH_W)], dst_all)
        plsc.subcore_barrier()

        pltpu.async_copy(g_hbm.at[src_all.at[0]], rows_a, sem_a)
        pltpu.async_copy(g_hbm.at[src_all.at[1]], rows_b, sem_b)

        def body(j, carry):
            ci = 2 * j
            pltpu.make_async_copy(g_hbm.at[pl.ds(0, CH)], rows_a, sem_a).wait()
            pltpu.sync_copy(rows_a, acc_sh.at[dst_all.at[ci]], add=True)
            pltpu.async_copy(g_hbm.at[src_all.at[ci + 2]], rows_a, sem_a)
            pltpu.make_async_copy(g_hbm.at[pl.ds(0, CH)], rows_b, sem_b).wait()
            pltpu.sync_copy(rows_b, acc_sh.at[dst_all.at[ci + 1]], add=True)
            pltpu.async_copy(g_hbm.at[src_all.at[ci + 3]], rows_b, sem_b)
            return carry

        lax.fori_loop(0, NCH_W // 2, body, 0)
        pltpu.make_async_copy(g_hbm.at[pl.ds(0, CH)], rows_a, sem_a).wait()
        pltpu.make_async_copy(g_hbm.at[pl.ds(0, CH)], rows_b, sem_b).wait()

        plsc.subcore_barrier()
        pltpu.sync_copy(acc_sh.at[pl.ds(s * RPS, RPS)],
                        out_hbm.at[c, pl.ds(s * RPS, RPS)])

    return _sc_agg


_sc_agg1 = _make_sc_agg(H1)
_sc_agg2 = _make_sc_agg(H2)


def _tc_g1_body(deg_ref, x_ref, w1_ref, g1_ref, dinv_ref):
    deg = deg_ref[0, :] + deg_ref[1, :] + 1.0
    dinv = lax.rsqrt(deg)
    dinv_ref[...] = dinv
    h = jnp.dot(x_ref[...], w1_ref[...], preferred_element_type=jnp.float32)
    g1_ref[:N, :] = h * dinv[:N, None]
    g1_ref[N:, :] = jnp.zeros((NP - N, H1), jnp.float32)


def _tc_g2_body(p0_ref, p1_ref, dinv_ref, b1_ref, w2_ref, g2_ref):
    dinv = dinv_ref[...][:N]
    a = p0_ref[:N, :] + p1_ref[:N, :]
    h = jnp.maximum(a * dinv[:, None] + b1_ref[...], 0.0)
    g2_ref[:N, :] = jnp.dot(h, w2_ref[...],
                            preferred_element_type=jnp.float32) * dinv[:, None]
    g2_ref[N:, :] = jnp.zeros((NP - N, H2), jnp.float32)


def _tc_h2_body(p0_ref, p1_ref, dinv_ref, b2_ref, h2_ref):
    dinv = dinv_ref[...][:N]
    a = p0_ref[:N, :] + p1_ref[:N, :]
    h2_ref[...] = jnp.maximum(a * dinv[:, None] + b2_ref[...], 0.0)


HR = (N * H2) // NW
HCH = 400
HNCH = HR // HCH
L = 16


@functools.partial(
    pl.kernel,
    out_type=jax.ShapeDtypeStruct((NW, ACT), jnp.float32),
    mesh=_MESH,
    compiler_params=_SC_PARAMS,
    scratch_types=[
        pltpu.VMEM((HR,), jnp.float32),
        pltpu.VMEM((HCH, ACT), jnp.float32),
        pltpu.VMEM((HCH, ACT), jnp.float32),
        pltpu.VMEM((ACT,), jnp.float32),
        pltpu.SemaphoreType.DMA,
        pltpu.SemaphoreType.DMA,
    ],
)
def _sc_head(w_hbm, flat_hbm, out_hbm, flat_v, wbuf_a, wbuf_b, part_v,
             sem_a, sem_b):
    c = lax.axis_index("c")
    s = lax.axis_index("s")
    w = s * NC + c
    base = w * HR

    pltpu.sync_copy(flat_hbm.at[pl.ds(base, HR)], flat_v)
    pltpu.async_copy(w_hbm.at[pl.ds(base, HCH)], wbuf_a, sem_a)
    pltpu.async_copy(w_hbm.at[pl.ds(base + HCH, HCH)], wbuf_b, sem_b)

    zero = jnp.zeros((L,), jnp.float32)

    def chunk(ci, buf, acc):
        off = ci * HCH

        def rows(g, acc):
            a0, a1, a2, a3 = acc
            fvec = flat_v[pl.ds(off + g * L, L)]
            for u in range(L):
                ri = g * L + u
                fv = fvec[u]
                a0 = a0 + fv * buf[ri, pl.ds(0, L)]
                a1 = a1 + fv * buf[ri, pl.ds(L, L)]
                a2 = a2 + fv * buf[ri, pl.ds(2 * L, L)]
                a3 = a3 + fv * buf[ri, pl.ds(3 * L, L)]
            return (a0, a1, a2, a3)

        return lax.fori_loop(0, HCH // L, rows, acc)

    def body(j, acc):
        ci = 2 * j
        pltpu.make_async_copy(w_hbm.at[pl.ds(0, HCH)], wbuf_a, sem_a).wait()
        acc = chunk(ci, wbuf_a, acc)

        @pl.when(ci + 2 < HNCH)
        def _():
            pltpu.async_copy(w_hbm.at[pl.ds(base + (ci + 2) * HCH, HCH)],
                             wbuf_a, sem_a)

        pltpu.make_async_copy(w_hbm.at[pl.ds(0, HCH)], wbuf_b, sem_b).wait()
        acc = chunk(ci + 1, wbuf_b, acc)

        @pl.when(ci + 3 < HNCH)
        def _():
            pltpu.async_copy(w_hbm.at[pl.ds(base + (ci + 3) * HCH, HCH)],
                             wbuf_b, sem_b)

        return acc

    acc = lax.fori_loop(0, HNCH // 2, body, (zero, zero, zero, zero))
    part_v[pl.ds(0, L)] = acc[0]
    part_v[pl.ds(L, L)] = acc[1]
    part_v[pl.ds(2 * L, L)] = acc[2]
    part_v[pl.ds(3 * L, L)] = acc[3]
    pltpu.sync_copy(part_v, out_hbm.at[w])


def _tc_softmax_body(p_ref, bout_ref, out_ref):
    logits = jnp.sum(p_ref[...], axis=0, keepdims=True) + bout_ref[...]
    m = jnp.max(logits, axis=-1, keepdims=True)
    e = jnp.exp(logits - m)
    out_ref[...] = e / jnp.sum(e, axis=-1, keepdims=True)


def kernel(x, ei, W1, b1, W2, b2, Wout, bout):
    src = ei[0].astype(jnp.int32)
    dst = ei[1].astype(jnp.int32)
    npad = EPA - E
    pad_idx = N + (jnp.arange(npad, dtype=jnp.int32) % (NP - N))
    src_p = jnp.concatenate([src, pad_idx]).reshape(NCHT, CH)
    dst_p = jnp.concatenate([dst, pad_idx]).reshape(NCHT, CH)

    zeros_np = jnp.zeros((NP,), jnp.float32)
    ones_ch = jnp.ones((CH,), jnp.float32)

    degp = _sc_deg(dst_p, zeros_np, ones_ch)

    g1, dinv = pl.pallas_call(
        _tc_g1_body,
        out_shape=[jax.ShapeDtypeStruct((NP, H1), jnp.float32),
                   jax.ShapeDtypeStruct((NP,), jnp.float32)],
    )(degp, x, W1)

    zeros_1 = jnp.zeros((NP, H1), jnp.float32)
    acc1 = _sc_agg1(g1, src_p, dst_p, zeros_1)

    g2 = pl.pallas_call(
        _tc_g2_body,
        out_shape=jax.ShapeDtypeStruct((NP, H2), jnp.float32),
    )(acc1[0], acc1[1], dinv, b1.reshape(1, H1), W2)

    zeros_2 = jnp.zeros((NP, H2), jnp.float32)
    acc2 = _sc_agg2(g2, src_p, dst_p, zeros_2)

    h2 = pl.pallas_call(
        _tc_h2_body,
        out_shape=jax.ShapeDtypeStruct((N, H2), jnp.float32),
    )(acc2[0][:N], acc2[1][:N], dinv, b2.reshape(1, H2))

    flat = h2.reshape(N * H2)
    parts = _sc_head(Wout, flat)
    probs = pl.pallas_call(
        _tc_softmax_body,
        out_shape=jax.ShapeDtypeStruct((1, ACT), jnp.float32),
    )(parts, bout.reshape(1, ACT))

    return probs


# speedup vs baseline: 1.4737x; 1.2526x over previous
import functools
import jax
import jax.numpy as jnp
from jax import lax
from jax.experimental import pallas as pl
from jax.experimental.pallas import tpu as pltpu
from jax.experimental.pallas import tpu_sc as plsc

N = 10000
NP = 10240
E = 320000
IN_DIM = 128
H1 = 32
H2 = 64
ACT = 64

NC, NS = 2, 16
NW = NC * NS
CH = 128
NCH_W = 80
NCHT = NW * NCH_W + 2
EPA = NCHT * CH
RPS = NP // NS

_MESH = plsc.VectorSubcoreMesh(core_axis_name="c", subcore_axis_name="s")
_SC_PARAMS = pltpu.CompilerParams(use_tc_tiling_on_sc=False)


@functools.partial(
    pl.kernel,
    out_type=jax.ShapeDtypeStruct((NC, NP), jnp.float32),
    mesh=_MESH,
    compiler_params=_SC_PARAMS,
    scratch_types=[
        pltpu.VMEM((NCH_W, CH), jnp.int32),
        pltpu.VMEM((CH,), jnp.float32),
        pltpu.VMEM_SHARED((NP,), jnp.float32),
    ],
)
def _sc_deg(dst_hbm, zeros_hbm, ones_hbm, out_hbm, dst_all, ones_v, acc_sh):
    c = lax.axis_index("c")
    s = lax.axis_index("s")
    w = s * NC + c

    pltpu.sync_copy(zeros_hbm.at[pl.ds(s * RPS, RPS)],
                    acc_sh.at[pl.ds(s * RPS, RPS)])
    pltpu.sync_copy(dst_hbm.at[pl.ds(w * NCH_W, NCH_W)], dst_all)
    pltpu.sync_copy(ones_hbm, ones_v)
    plsc.subcore_barrier()

    def body(i, carry):
        pltpu.sync_copy(ones_v, acc_sh.at[dst_all.at[i]], add=True)
        return carry

    lax.fori_loop(0, NCH_W, body, 0)
    plsc.subcore_barrier()
    pltpu.sync_copy(acc_sh.at[pl.ds(s * RPS, RPS)],
                    out_hbm.at[c, pl.ds(s * RPS, RPS)])


def _make_sc_agg(F):
    @functools.partial(
        pl.kernel,
        out_type=jax.ShapeDtypeStruct((NC, NP, F), jnp.float32),
        mesh=_MESH,
        compiler_params=_SC_PARAMS,
        scratch_types=[
            pltpu.VMEM((NCH_W + 2, CH), jnp.int32),
            pltpu.VMEM((NCH_W, CH), jnp.int32),
            pltpu.VMEM((CH, F), jnp.float32),
            pltpu.VMEM((CH, F), jnp.float32),
            pltpu.VMEM_SHARED((NP, F), jnp.float32),
            pltpu.SemaphoreType.DMA,
            pltpu.SemaphoreType.DMA,
        ],
    )
    def _sc_agg(g_hbm, src_hbm, dst_hbm, zeros_hbm, out_hbm,
                src_all, dst_all, rows_a, rows_b, acc_sh, sem_a, sem_b):
        c = lax.axis_index("c")
        s = lax.axis_index("s")
        w = s * NC + c

        rr = pl.ds(s * RPS, RPS)

        @pl.when(c == 0)
        def _():
            pltpu.sync_copy(g_hbm.at[rr], acc_sh.at[rr])

        @pl.when(c == 1)
        def _():
            pltpu.sync_copy(zeros_hbm.at[rr], acc_sh.at[rr])

        base = w * NCH_W
        pltpu.sync_copy(src_hbm.at[pl.ds(base, NCH_W + 2)], src_all)
        pltpu.sync_copy(dst_hbm.at[pl.ds(base, NCH_W)], dst_all)
        plsc.subcore_barrier()

        pltpu.async_copy(g_hbm.at[src_all.at[0]], rows_a, sem_a)
        pltpu.async_copy(g_hbm.at[src_all.at[1]], rows_b, sem_b)

        def body(j, carry):
            ci = 2 * j
            pltpu.make_async_copy(g_hbm.at[pl.ds(0, CH)], rows_a, sem_a).wait()
            pltpu.sync_copy(rows_a, acc_sh.at[dst_all.at[ci]], add=True)
            pltpu.async_copy(g_hbm.at[src_all.at[ci + 2]], rows_a, sem_a)
            pltpu.make_async_copy(g_hbm.at[pl.ds(0, CH)], rows_b, sem_b).wait()
            pltpu.sync_copy(rows_b, acc_sh.at[dst_all.at[ci + 1]], add=True)
            pltpu.async_copy(g_hbm.at[src_all.at[ci + 3]], rows_b, sem_b)
            return carry

        lax.fori_loop(0, NCH_W // 2, body, 0)
        pltpu.make_async_copy(g_hbm.at[pl.ds(0, CH)], rows_a, sem_a).wait()
        pltpu.make_async_copy(g_hbm.at[pl.ds(0, CH)], rows_b, sem_b).wait()

        plsc.subcore_barrier()
        pltpu.sync_copy(acc_sh.at[pl.ds(s * RPS, RPS)],
                        out_hbm.at[c, pl.ds(s * RPS, RPS)])

    return _sc_agg


_sc_agg1 = _make_sc_agg(H1)
_sc_agg2 = _make_sc_agg(H2)


def _tc_g1_body(deg_ref, x_ref, w1_ref, g1_ref, dinv_ref):
    deg = deg_ref[0, :] + deg_ref[1, :] + 1.0
    dinv = lax.rsqrt(deg)
    dinv_ref[...] = dinv
    h = jnp.dot(x_ref[...], w1_ref[...], preferred_element_type=jnp.float32)
    g1_ref[:N, :] = h * dinv[:N, None]
    g1_ref[N:, :] = jnp.zeros((NP - N, H1), jnp.float32)


def _tc_g2_body(acc_ref, dinv_ref, b1_ref, w2_ref, g2_ref):
    dinv = dinv_ref[...][:N]
    a = acc_ref[0, :N, :] + acc_ref[1, :N, :]
    h = jnp.maximum(a * dinv[:, None] + b1_ref[...][None, :], 0.0)
    g2_ref[:N, :] = jnp.dot(h, w2_ref[...],
                            preferred_element_type=jnp.float32) * dinv[:, None]
    g2_ref[N:, :] = jnp.zeros((NP - N, H2), jnp.float32)


def _tc_h2_body(acc_ref, dinv_ref, b2_ref, h2_ref):
    dinv = dinv_ref[...][:N]
    a = acc_ref[0, :N, :] + acc_ref[1, :N, :]
    h2_ref[...] = jnp.maximum(a * dinv[:, None] + b2_ref[...][None, :], 0.0)


BK = 12800
KSTEPS = (N * H2) // BK


def _tc_head_body(a_ref, w_ref, bout_ref, out_ref):
    k = pl.program_id(0)
    part = jnp.dot(a_ref[...], w_ref[...], preferred_element_type=jnp.float32)

    @pl.when(k == 0)
    def _():
        out_ref[...] = part

    @pl.when(k > 0)
    def _():
        out_ref[...] += part

    @pl.when(k == KSTEPS - 1)
    def _():
        logits = out_ref[...] + bout_ref[...]
        m = jnp.max(logits, axis=-1, keepdims=True)
        e = jnp.exp(logits - m)
        out_ref[...] = e / jnp.sum(e, axis=-1, keepdims=True)


def kernel(x, ei, W1, b1, W2, b2, Wout, bout):
    src = ei[0].astype(jnp.int32)
    dst = ei[1].astype(jnp.int32)
    npad = EPA - E
    pad_idx = N + (jnp.arange(npad, dtype=jnp.int32) % (NP - N))
    src_p = jnp.concatenate([src, pad_idx]).reshape(NCHT, CH)
    dst_p = jnp.concatenate([dst, pad_idx]).reshape(NCHT, CH)

    zeros_np = jnp.zeros((NP,), jnp.float32)
    ones_ch = jnp.ones((CH,), jnp.float32)

    degp = _sc_deg(dst_p, zeros_np, ones_ch)

    g1, dinv = pl.pallas_call(
        _tc_g1_body,
        out_shape=[jax.ShapeDtypeStruct((NP, H1), jnp.float32),
                   jax.ShapeDtypeStruct((NP,), jnp.float32)],
    )(degp, x, W1)

    zeros_1 = jnp.zeros((NP, H1), jnp.float32)
    acc1 = _sc_agg1(g1, src_p, dst_p, zeros_1)

    g2 = pl.pallas_call(
        _tc_g2_body,
        out_shape=jax.ShapeDtypeStruct((NP, H2), jnp.float32),
    )(acc1, dinv, b1, W2)

    zeros_2 = jnp.zeros((NP, H2), jnp.float32)
    acc2 = _sc_agg2(g2, src_p, dst_p, zeros_2)

    h2 = pl.pallas_call(
        _tc_h2_body,
        out_shape=jax.ShapeDtypeStruct((N, H2), jnp.float32),
    )(acc2, dinv, b2)

    flat = h2.reshape(1, N * H2)
    probs = pl.pallas_call(
        _tc_head_body,
        grid=(KSTEPS,),
        in_specs=[
            pl.BlockSpec((1, BK), lambda k: (0, k)),
            pl.BlockSpec((BK, ACT), lambda k: (k, 0)),
            pl.BlockSpec((1, ACT), lambda k: (0, 0)),
        ],
        out_specs=pl.BlockSpec((1, ACT), lambda k: (0, 0)),
        out_shape=jax.ShapeDtypeStruct((1, ACT), jnp.float32),
    )(flat, Wout, bout.reshape(1, ACT))

    return probs
